# baseline (speedup 1.0000x reference)
import functools

import jax
import jax.numpy as jnp
import numpy as np
from jax.experimental import pallas as pl
from jax.experimental.pallas import tpu as pltpu

HEADS = 4
CH = 16
HC = HEADS * CH


def _mm_kernel(x_ref, w_ref, b_ref, o_ref):
    o_ref[...] = (
        jnp.dot(x_ref[...], w_ref[...], preferred_element_type=jnp.float32)
        + b_ref[...]
    )


def _project(x, W, b, blk):
    M, K = x.shape
    N = W.shape[1]
    assert M % blk == 0, (M, blk)
    return pl.pallas_call(
        _mm_kernel,
        grid=(M // blk,),
        in_specs=[
            pl.BlockSpec((blk, K), lambda i: (i, 0)),
            pl.BlockSpec((K, N), lambda i: (0, 0)),
            pl.BlockSpec((1, N), lambda i: (0, 0)),
        ],
        out_specs=pl.BlockSpec((blk, N), lambda i: (i, 0)),
        out_shape=jax.ShapeDtypeStruct((M, N), jnp.float32),
    )(x, W, b.reshape(1, N))


def _cat_params(p):
    W = jnp.concatenate([p["q"]["W"], p["k"]["W"], p["v"]["W"], p["skip"]["W"]], axis=1)
    b = jnp.concatenate([p["q"]["b"], p["k"]["b"], p["v"]["b"], p["skip"]["b"]])
    return W, b


def _conv(x, src, dst, e, params, N):
    qkvs = _project(x, *_cat_params(params), blk=1000)
    q = qkvs[:, 0:64].reshape(N, HEADS, CH)[dst]
    k = qkvs[:, 64:128].reshape(N, HEADS, CH)[src]
    v = qkvs[:, 128:192].reshape(N, HEADS, CH)[src]
    skip = qkvs[:, 192:256]
    e3 = e.reshape(-1, HEADS, CH)
    k = k + e3
    alpha = (q * k).sum(-1) / np.sqrt(CH)
    amax = jax.ops.segment_max(alpha, dst, num_segments=N)
    amax = jnp.where(jnp.isfinite(amax), amax, 0.0)
    ex = jnp.exp(alpha - amax[dst])
    denom = jax.ops.segment_sum(ex, dst, num_segments=N)
    a = ex / denom[dst]
    out = jax.ops.segment_sum((v + e3) * a[..., None], dst, num_segments=N)
    return out.reshape(N, HC) + skip


def kernel(memory, n_id, edge_index, edge_attr, params1, params2):
    N = n_id.shape[0]
    x = memory[n_id]
    src, dst = edge_index[0], edge_index[1]
    We = jnp.concatenate([params1["e"]["W"], params2["e"]["W"]], axis=1)
    be = jnp.concatenate([params1["e"]["b"], params2["e"]["b"]])
    e12 = _project(edge_attr, We, be, blk=2000)
    x = jax.nn.relu(_conv(x, src, dst, e12[:, 0:64], params1, N))
    x = _conv(x, src, dst, e12[:, 64:128], params2, N)
    return x


# speedup vs baseline: 12.6431x; 12.6431x over previous
import functools

import jax
import jax.numpy as jnp
import numpy as np
from jax import lax
from jax.experimental import pallas as pl
from jax.experimental.pallas import tpu as pltpu
from jax.experimental.pallas import tpu_sc as plsc

HEADS = 4
CH = 16
HC = HEADS * CH

NC, NS = 2, 16
NT = NC * NS

_MESH = plsc.VectorSubcoreMesh(
    core_axis_name="c", subcore_axis_name="s", num_cores=NC, num_subcores=NS
)
_SC_PARAMS = pltpu.CompilerParams(
    use_tc_tiling_on_sc=False, needs_layout_passes=False
)


def _wid():
    return lax.axis_index("s") * NC + lax.axis_index("c")


def _gather_rows(table, idx, rows_per_tile=1664, chunk=128):
    B = idx.shape[0]
    D = table.shape[1]
    assert B == NT * rows_per_tile and rows_per_tile % chunk == 0

    @functools.partial(
        pl.kernel,
        out_type=jax.ShapeDtypeStruct((B, D), jnp.float32),
        mesh=_MESH,
        compiler_params=_SC_PARAMS,
        scratch_types=[
            pltpu.VMEM((chunk,), jnp.int32),
            pltpu.VMEM((chunk, D), jnp.float32),
            pltpu.SemaphoreType.DMA,
        ],
    )
    def k(table_hbm, idx_hbm, out_hbm, idx_v, rows_v, sem):
        base = _wid() * rows_per_tile

        def body(c, _):
            off = base + c * chunk
            pltpu.sync_copy(idx_hbm.at[pl.ds(off, chunk)], idx_v)
            pltpu.async_copy(table_hbm.at[idx_v], rows_v, sem).wait()
            pltpu.sync_copy(rows_v, out_hbm.at[pl.ds(off, chunk)])
            return 0

        lax.fori_loop(0, rows_per_tile // chunk, body, 0)

    return k(table, idx)


E = 800000
NRND = 2
NB = NT * NRND
BW = 784
ES = E // NT
CAP = 25088
ECH = 128
YW = 80
DUMP = BW
RSTR = BW + 16
ALEN = (E + NT * NB * 128) * 4


def _bucket_edges(src, dst):
    @functools.partial(
        pl.kernel,
        out_type=(
            jax.ShapeDtypeStruct((NT, NB, CAP), jnp.int32),
            jax.ShapeDtypeStruct((NT, NB, CAP), jnp.int32),
            jax.ShapeDtypeStruct((NT, NB, CAP), jnp.int32),
            jax.ShapeDtypeStruct((NT * NB,), jnp.int32),
        ),
        mesh=_MESH,
        compiler_params=_SC_PARAMS,
        scratch_types=[
            pltpu.VMEM((1000,), jnp.int32),
            pltpu.VMEM((1000,), jnp.int32),
            pltpu.VMEM((NB * 64,), jnp.int32),
            pltpu.VMEM((NB * 64,), jnp.int32),
            pltpu.VMEM((NB * 64,), jnp.int32),
            pltpu.VMEM((NB,), jnp.int32),
        ],
    )
    def k(src_hbm, dst_hbm, eid_out, src_out, dst_out, cnt_out,
          sbuf, dbuf, st_e, st_s, st_d, cntv):
        w = _wid()
        base_e = w * ES
        iota = lax.iota(jnp.int32, 16)
        m0 = iota == 0

        def _sget(ref, i):
            return plsc.load_gather(ref, [jnp.full((16,), i, jnp.int32)])[0]

        def _sput(ref, i, v):
            plsc.store_scatter(ref, [jnp.full((16,), i, jnp.int32)],
                               jnp.full((16,), v), mask=m0)

        def zb(g, _):
            cntv[pl.ds(g * 16, 16)] = jnp.zeros((16,), jnp.int32)
            return 0

        lax.fori_loop(0, NB // 16, zb, 0)

        def chunk(c, _):
            off = c * 1000
            pltpu.sync_copy(src_hbm.at[pl.ds(base_e + off, 1000)], sbuf)
            pltpu.sync_copy(dst_hbm.at[pl.ds(base_e + off, 1000)], dbuf)

            def edge(j, _):
                d = _sget(dbuf, j)
                b = d // BW
                cb = _sget(cntv, b)
                slot = cb & 63
                _sput(st_e, b * 64 + slot, base_e + off + j)
                _sput(st_s, b * 64 + slot, _sget(sbuf, j))
                _sput(st_d, b * 64 + slot, d)

                @pl.when(slot == 63)
                def _flush():
                    fo = pl.multiple_of(cb - 63, 64)
                    pltpu.sync_copy(st_e.at[pl.ds(b * 64, 64)],
                                    eid_out.at[w, b, pl.ds(fo, 64)])
                    pltpu.sync_copy(st_s.at[pl.ds(b * 64, 64)],
                                    src_out.at[w, b, pl.ds(fo, 64)])
                    pltpu.sync_copy(st_d.at[pl.ds(b * 64, 64)],
                                    dst_out.at[w, b, pl.ds(fo, 64)])

                _sput(cntv, b, cb + 1)
                return 0

            lax.fori_loop(0, 1000, edge, 0)
            return 0

        lax.fori_loop(0, ES // 1000, chunk, 0)

        def drain(b, _):
            cb = _sget(cntv, b)

            @pl.when((cb & 63) != 0)
            def _fl():
                fo = pl.multiple_of(cb - (cb & 63), 64)
                pltpu.sync_copy(st_e.at[pl.ds(b * 64, 64)],
                                eid_out.at[w, b, pl.ds(fo, 64)])
                pltpu.sync_copy(st_s.at[pl.ds(b * 64, 64)],
                                src_out.at[w, b, pl.ds(fo, 64)])
                pltpu.sync_copy(st_d.at[pl.ds(b * 64, 64)],
                                dst_out.at[w, b, pl.ds(fo, 64)])

            return 0

        lax.fori_loop(0, NB, drain, 0)
        pltpu.sync_copy(cntv, cnt_out.at[pl.ds(w * NB, NB)])

    return k(src, dst)


def _attn_layer_fn(relu):
    @functools.partial(
        pl.kernel,
        out_type=(
            jax.ShapeDtypeStruct((NB * BW, HC), jnp.float32),
            jax.ShapeDtypeStruct((ALEN,), jnp.float32),
        ),
        mesh=_MESH,
        compiler_params=_SC_PARAMS,
        scratch_types=[
            pltpu.VMEM((ECH,), jnp.int32),
            pltpu.VMEM((ECH,), jnp.int32),
            pltpu.VMEM((ECH,), jnp.int32),
            pltpu.VMEM((ECH,), jnp.int32),
            pltpu.VMEM((ECH, HC), jnp.float32),
            pltpu.VMEM((ECH, HC), jnp.float32),
            pltpu.VMEM((ECH, HC), jnp.float32),
            pltpu.VMEM((ECH * 4,), jnp.float32),
            pltpu.VMEM((ECH * 4,), jnp.float32),
            pltpu.VMEM((ECH, YW), jnp.float32),
            pltpu.VMEM((BW * 4 + 16,), jnp.float32),
            pltpu.VMEM((NT * NB,), jnp.int32),
            pltpu.VMEM_SHARED((NS * RSTR, YW), jnp.float32),
            pltpu.SemaphoreType.DMA,
            pltpu.SemaphoreType.DMA,
            pltpu.SemaphoreType.DMA,
        ],
    )
    def k(qn_h, kn_h, vn_h, skipn_h, etab_h, eid_h, src_h, dst_h, cnt_h,
          out_h, alpha_h,
          eidb, srcb, dstb, dstlb, qvb, kb, eb, alphab, exb, contrib,
          amax, cntsv, ysh, sem1, sem2, sem3):
        cidx = lax.axis_index("c")
        sidx = lax.axis_index("s")
        w = sidx * NC + cidx
        iota = lax.iota(jnp.int32, 16)
        m0 = iota == 0
        i4f = iota // 4
        i4m = iota - i4f * 4

        def _sget(ref, i):
            return plsc.load_gather(ref, [jnp.full((16,), i, jnp.int32)])[0]

        def _sputf(ref, i, v):
            plsc.store_scatter(ref, [jnp.full((16,), i, jnp.int32)],
                               jnp.full((16,), v), mask=m0)

        pltpu.sync_copy(cnt_h, cntsv)

        def _round(bkt):
            base_n = bkt * BW
            nloc = jnp.minimum(BW, 50000 - base_n)

            def gs_b(b, acc):
                def gs_t(t, a):
                    return a + ((_sget(cntsv, t * NB + b) + 127) // 128) * 128

                tot = lax.fori_loop(0, NT, gs_t, 0)
                return acc + jnp.where(b < bkt, tot, 0)

            gstart = lax.fori_loop(0, NB, gs_b, 0)

            def zc(r, _):
                for qq in range(YW // 16):
                    contrib[r, pl.ds(qq * 16, 16)] = jnp.zeros((16,), jnp.float32)
                return 0

            lax.fori_loop(0, ECH, zc, 0)

            def zy(cix, _):
                pltpu.sync_copy(
                    contrib,
                    ysh.at[pl.ds(
                        sidx * RSTR + pl.multiple_of(cix * ECH, ECH), ECH)])
                return 0

            lax.fori_loop(0, BW // ECH, zy, 0)

            def zy2(cix, _):
                pltpu.sync_copy(
                    contrib.at[pl.ds(0, 16)],
                    ysh.at[pl.ds(
                        sidx * RSTR
                        + pl.multiple_of((BW // ECH) * ECH + cix * 16, 16), 16)])
                return 0

            lax.fori_loop(0, (BW % ECH) // 16, zy2, 0)

            neg = jnp.full((16,), -3.0e38, jnp.float32)

            def za(g, _):
                amax[pl.ds(pl.multiple_of(g * 16, 16), 16)] = neg
                return 0

            lax.fori_loop(0, (BW * 4 + 16) // 16, za, 0)

            def p1_t(t, done):
                cnt = _sget(cntsv, t * NB + bkt)
                nch = (cnt + 127) // 128

                def p1_c(ch, _):
                    off = pl.multiple_of(ch * ECH, ECH)
                    lv = jnp.minimum(ECH, cnt - off)
                    pltpu.sync_copy(eid_h.at[t, bkt, pl.ds(off, ECH)], eidb)
                    pltpu.sync_copy(src_h.at[t, bkt, pl.ds(off, ECH)], srcb)
                    pltpu.sync_copy(dst_h.at[t, bkt, pl.ds(off, ECH)], dstb)

                    def san(g, _):
                        ssl = pl.ds(pl.multiple_of(g * 16, 16), 16)
                        msk = (g * 16 + iota) < lv
                        eidb[ssl] = jnp.where(msk, eidb[ssl], 0)
                        srcb[ssl] = jnp.where(msk, srcb[ssl], 0)
                        dstb[ssl] = jnp.where(msk, dstb[ssl], base_n)
                        return 0

                    lax.fori_loop(0, ECH // 16, san, 0)

                    c1 = pltpu.async_copy(qn_h.at[dstb], qvb, sem1)
                    c2 = pltpu.async_copy(kn_h.at[srcb], kb, sem2)
                    c3 = pltpu.async_copy(etab_h.at[eidb], eb, sem3)
                    c1.wait()
                    c2.wait()
                    c3.wait()

                    def p1_e(j, _):
                        dl = _sget(dstb, j) - base_n
                        for h in range(HEADS):
                            sl = pl.ds(h * CH, CH)
                            s_h = jnp.sum(
                                qvb[j, sl] * (kb[j, sl] + eb[j, sl])) * 0.25
                            ai = dl * 4 + h
                            mold = _sget(amax, ai)
                            _sputf(amax, ai, jnp.maximum(mold, s_h))
                            _sputf(alphab, j * 4 + h, s_h)
                        return 0

                    lax.fori_loop(0, lv, p1_e, 0)
                    pltpu.sync_copy(
                        alphab,
                        alpha_h.at[pl.ds(
                            pl.multiple_of((gstart + done + off) * 4, ECH * 4),
                            ECH * 4)])
                    return 0

                lax.fori_loop(0, nch, p1_c, 0)
                return done + nch * ECH

            lax.fori_loop(0, NT, p1_t, 0)

            def p2_t(t, done):
                cnt = _sget(cntsv, t * NB + bkt)
                nch = (cnt + 127) // 128

                def p2_c(ch, _):
                    off = pl.multiple_of(ch * ECH, ECH)
                    lv = jnp.minimum(ECH, cnt - off)
                    pltpu.sync_copy(eid_h.at[t, bkt, pl.ds(off, ECH)], eidb)
                    pltpu.sync_copy(src_h.at[t, bkt, pl.ds(off, ECH)], srcb)
                    pltpu.sync_copy(dst_h.at[t, bkt, pl.ds(off, ECH)], dstb)

                    def san(g, _):
                        ssl = pl.ds(pl.multiple_of(g * 16, 16), 16)
                        msk = (g * 16 + iota) < lv
                        eidb[ssl] = jnp.where(msk, eidb[ssl], 0)
                        srcb[ssl] = jnp.where(msk, srcb[ssl], 0)
                        dstb[ssl] = jnp.where(msk, dstb[ssl], base_n)
                        return 0

                    lax.fori_loop(0, ECH // 16, san, 0)

                    c1 = pltpu.async_copy(vn_h.at[srcb], qvb, sem1)
                    c2 = pltpu.async_copy(etab_h.at[eidb], eb, sem2)
                    pltpu.sync_copy(
                        alpha_h.at[pl.ds(
                            pl.multiple_of((gstart + done + off) * 4, ECH * 4),
                            ECH * 4)], alphab)

                    def mkdl(g, _):
                        sl = pl.ds(pl.multiple_of(g * 16, 16), 16)
                        lanes = g * 16 + iota
                        dv = dstb[sl] - base_n
                        dstlb[sl] = sidx * RSTR + jnp.where(
                            lanes < lv, dv, DUMP)
                        return 0

                    lax.fori_loop(0, ECH // 16, mkdl, 0)

                    def mkex(g, _):
                        ev = plsc.load_gather(dstlb, [g * 4 + i4f]) - sidx * RSTR
                        av = alphab[pl.ds(pl.multiple_of(g * 16, 16), 16)]
                        mg = plsc.load_gather(amax, [ev * 4 + i4m])
                        exv = jnp.where(ev < DUMP, jnp.exp(av - mg), 0.0)
                        exb[pl.ds(pl.multiple_of(g * 16, 16), 16)] = exv
                        plsc.store_scatter(contrib, [g * 4 + i4f, 64 + i4m], exv)
                        return 0

                    lax.fori_loop(0, ECH * 4 // 16, mkex, 0)
                    c1.wait()
                    c2.wait()

                    def p2_e(j, _):
                        exq = plsc.load_gather(exb, [j * 4 + (iota & 3)])
                        for h in range(HEADS):
                            sl = pl.ds(h * CH, CH)
                            contrib[j, sl] = (qvb[j, sl] + eb[j, sl]) * exq[h]
                        return 0

                    lax.fori_loop(0, lv, p2_e, 0)
                    pltpu.sync_copy(contrib, ysh.at[dstlb], add=True)
                    return 0

                lax.fori_loop(0, nch, p2_c, 0)
                return done + nch * ECH

            lax.fori_loop(0, NT, p2_t, 0)

            def p3_n(nn, _):
                denv = contrib[nn, pl.ds(64, 16)]
                rv = jnp.where(denv != 0.0, 1.0 / denv, 0.0)
                for h in range(HEADS):
                    sl = pl.ds(h * CH, CH)
                    ov = contrib[nn, sl] * rv[h] + kb[nn, sl]
                    if relu:
                        ov = jnp.maximum(ov, 0.0)
                    eb[nn, sl] = ov
                return 0

            nfull = nloc // ECH
            ntail = (nloc - nfull * ECH) // 16

            def p3_f(cix, _):
                off = pl.multiple_of(cix * ECH, ECH)
                pltpu.sync_copy(ysh.at[pl.ds(sidx * RSTR + off, ECH)], contrib)
                pltpu.sync_copy(skipn_h.at[pl.ds(base_n + off, ECH)], kb)
                lax.fori_loop(0, ECH, p3_n, 0)
                pltpu.sync_copy(eb, out_h.at[pl.ds(base_n + off, ECH)])
                return 0

            lax.fori_loop(0, nfull, p3_f, 0)

            def p3_t(cix, _):
                off = pl.multiple_of(nfull * ECH + cix * 16, 16)
                pltpu.sync_copy(ysh.at[pl.ds(sidx * RSTR + off, 16)],
                                contrib.at[pl.ds(0, 16)])
                pltpu.sync_copy(skipn_h.at[pl.ds(base_n + off, 16)],
                                kb.at[pl.ds(0, 16)])
                lax.fori_loop(0, 16, p3_n, 0)
                pltpu.sync_copy(eb.at[pl.ds(0, 16)],
                                out_h.at[pl.ds(base_n + off, 16)])
                return 0

            lax.fori_loop(0, ntail, p3_t, 0)

        for rnd in range(NRND):
            _round(w + NT * rnd)

    return k


_attn_relu = _attn_layer_fn(True)
_attn_plain = _attn_layer_fn(False)


def _mm_kernel(x_ref, w_ref, b_ref, o_ref):
    o_ref[...] = (
        jnp.dot(x_ref[...], w_ref[...], preferred_element_type=jnp.float32)
        + b_ref[...]
    )


def _project(x, W, b, blk):
    M, K = x.shape
    N = W.shape[1]
    assert M % blk == 0, (M, blk)
    return pl.pallas_call(
        _mm_kernel,
        grid=(M // blk,),
        in_specs=[
            pl.BlockSpec((blk, K), lambda i: (i, 0)),
            pl.BlockSpec((K, N), lambda i: (0, 0)),
            pl.BlockSpec((1, N), lambda i: (0, 0)),
        ],
        out_specs=pl.BlockSpec((blk, N), lambda i: (i, 0)),
        out_shape=jax.ShapeDtypeStruct((M, N), jnp.float32),
    )(x, W, b.reshape(1, N))


def _mm_split_kernel(nout, x_ref, w_ref, b_ref, *o_refs):
    acc = (
        jnp.dot(x_ref[...], w_ref[...], preferred_element_type=jnp.float32)
        + b_ref[...]
    )
    for i, o in enumerate(o_refs):
        o[...] = acc[:, i * HC:(i + 1) * HC]


def _project_split(x, Ws, bs, blk):
    M, K = x.shape
    nout = len(Ws)
    W = jnp.concatenate(Ws, axis=1)
    b = jnp.concatenate(bs).reshape(1, nout * HC)
    assert M % blk == 0, (M, blk)
    return pl.pallas_call(
        functools.partial(_mm_split_kernel, nout),
        grid=(M // blk,),
        in_specs=[
            pl.BlockSpec((blk, K), lambda i: (i, 0)),
            pl.BlockSpec((K, nout * HC), lambda i: (0, 0)),
            pl.BlockSpec((1, nout * HC), lambda i: (0, 0)),
        ],
        out_specs=[
            pl.BlockSpec((blk, HC), lambda i: (i, 0)) for _ in range(nout)
        ],
        out_shape=[
            jax.ShapeDtypeStruct((M, HC), jnp.float32) for _ in range(nout)
        ],
    )(x, W, b)


def _proj_nodes(x, p):
    return _project_split(
        x,
        [p["q"]["W"], p["k"]["W"], p["v"]["W"], p["skip"]["W"]],
        [p["q"]["b"], p["k"]["b"], p["v"]["b"], p["skip"]["b"]],
        blk=512,
    )


def kernel(memory, n_id, edge_index, edge_attr, params1, params2):
    N = n_id.shape[0]
    nid_pad = jnp.pad(n_id.astype(jnp.int32), (0, NT * 1664 - N))
    x = _gather_rows(memory, nid_pad)[: NB * BW]
    src = edge_index[0].astype(jnp.int32)
    dst = edge_index[1].astype(jnp.int32)
    eid_r, src_r, dst_r, counts = _bucket_edges(src, dst)
    e1, e2 = _project_split(
        edge_attr,
        [params1["e"]["W"], params2["e"]["W"]],
        [params1["e"]["b"], params2["e"]["b"]],
        blk=2000,
    )
    qn, kn, vn, sk = _proj_nodes(x, params1)
    x2, _ = _attn_relu(qn, kn, vn, sk, e1, eid_r, src_r, dst_r, counts)
    qn2, kn2, vn2, sk2 = _proj_nodes(x2, params2)
    x3, _ = _attn_plain(qn2, kn2, vn2, sk2, e2, eid_r, src_r, dst_r, counts)
    return x3[:N]


# speedup vs baseline: 13.6057x; 1.0761x over previous
import functools

import jax
import jax.numpy as jnp
import numpy as np
from jax import lax
from jax.experimental import pallas as pl
from jax.experimental.pallas import tpu as pltpu
from jax.experimental.pallas import tpu_sc as plsc

HEADS = 4
CH = 16
HC = HEADS * CH

NC, NS = 2, 16
NT = NC * NS

_MESH = plsc.VectorSubcoreMesh(
    core_axis_name="c", subcore_axis_name="s", num_cores=NC, num_subcores=NS
)
_SC_PARAMS = pltpu.CompilerParams(
    use_tc_tiling_on_sc=False, needs_layout_passes=False
)


def _wid():
    return lax.axis_index("s") * NC + lax.axis_index("c")


def _gather_rows(table, idx, rows_per_tile=1664, chunk=128):
    B = idx.shape[0]
    D = table.shape[1]
    assert B == NT * rows_per_tile and rows_per_tile % chunk == 0

    @functools.partial(
        pl.kernel,
        out_type=jax.ShapeDtypeStruct((B, D), jnp.float32),
        mesh=_MESH,
        compiler_params=_SC_PARAMS,
        scratch_types=[
            pltpu.VMEM((chunk,), jnp.int32),
            pltpu.VMEM((chunk, D), jnp.float32),
            pltpu.SemaphoreType.DMA,
        ],
    )
    def k(table_hbm, idx_hbm, out_hbm, idx_v, rows_v, sem):
        base = _wid() * rows_per_tile

        def body(c, _):
            off = base + c * chunk
            pltpu.sync_copy(idx_hbm.at[pl.ds(off, chunk)], idx_v)
            pltpu.async_copy(table_hbm.at[idx_v], rows_v, sem).wait()
            pltpu.sync_copy(rows_v, out_hbm.at[pl.ds(off, chunk)])
            return 0

        lax.fori_loop(0, rows_per_tile // chunk, body, 0)

    return k(table, idx)


E = 800000
NRND = 2
NB = NT * NRND
BW = 784
ES = E // NT
CAP = 25088
ECH = 128
YW = 80
DUMP = BW
RSTR = BW + 16
ALEN = (E + NT * NB * ECH) * 4


def _bucket_edges(src, dst):
    @functools.partial(
        pl.kernel,
        out_type=(
            jax.ShapeDtypeStruct((NT, NB, CAP), jnp.int32),
            jax.ShapeDtypeStruct((NT, NB, CAP), jnp.int32),
            jax.ShapeDtypeStruct((NT, NB, CAP), jnp.int32),
            jax.ShapeDtypeStruct((NT * NB,), jnp.int32),
        ),
        mesh=_MESH,
        compiler_params=_SC_PARAMS,
        scratch_types=[
            pltpu.VMEM((1000,), jnp.int32),
            pltpu.VMEM((1000,), jnp.int32),
            pltpu.VMEM((NB * 64,), jnp.int32),
            pltpu.VMEM((NB * 64,), jnp.int32),
            pltpu.VMEM((NB * 64,), jnp.int32),
            pltpu.VMEM((NB,), jnp.int32),
        ],
    )
    def k(src_hbm, dst_hbm, eid_out, src_out, dst_out, cnt_out,
          sbuf, dbuf, st_e, st_s, st_d, cntv):
        w = _wid()
        base_e = w * ES
        iota = lax.iota(jnp.int32, 16)
        m0 = iota == 0

        def _sget(ref, i):
            return plsc.load_gather(ref, [jnp.full((16,), i, jnp.int32)])[0]

        def _sput(ref, i, v):
            plsc.store_scatter(ref, [jnp.full((16,), i, jnp.int32)],
                               jnp.full((16,), v), mask=m0)

        def zb(g, _):
            cntv[pl.ds(g * 16, 16)] = jnp.zeros((16,), jnp.int32)
            return 0

        lax.fori_loop(0, NB // 16, zb, 0)

        def chunk(c, _):
            off = c * 1000
            pltpu.sync_copy(src_hbm.at[pl.ds(base_e + off, 1000)], sbuf)
            pltpu.sync_copy(dst_hbm.at[pl.ds(base_e + off, 1000)], dbuf)

            def edge(j, _):
                d = _sget(dbuf, j)
                b = d // BW
                cb = _sget(cntv, b)
                slot = cb & 63
                _sput(st_e, b * 64 + slot, base_e + off + j)
                _sput(st_s, b * 64 + slot, _sget(sbuf, j))
                _sput(st_d, b * 64 + slot, d)

                @pl.when(slot == 63)
                def _flush():
                    fo = pl.multiple_of(cb - 63, 64)
                    pltpu.sync_copy(st_e.at[pl.ds(b * 64, 64)],
                                    eid_out.at[w, b, pl.ds(fo, 64)])
                    pltpu.sync_copy(st_s.at[pl.ds(b * 64, 64)],
                                    src_out.at[w, b, pl.ds(fo, 64)])
                    pltpu.sync_copy(st_d.at[pl.ds(b * 64, 64)],
                                    dst_out.at[w, b, pl.ds(fo, 64)])

                _sput(cntv, b, cb + 1)
                return 0

            lax.fori_loop(0, 1000, edge, 0)
            return 0

        lax.fori_loop(0, ES // 1000, chunk, 0)

        def drain(b, _):
            cb = _sget(cntv, b)

            @pl.when((cb & 63) != 0)
            def _fl():
                fo = pl.multiple_of(cb - (cb & 63), 64)
                pltpu.sync_copy(st_e.at[pl.ds(b * 64, 64)],
                                eid_out.at[w, b, pl.ds(fo, 64)])
                pltpu.sync_copy(st_s.at[pl.ds(b * 64, 64)],
                                src_out.at[w, b, pl.ds(fo, 64)])
                pltpu.sync_copy(st_d.at[pl.ds(b * 64, 64)],
                                dst_out.at[w, b, pl.ds(fo, 64)])

            return 0

        lax.fori_loop(0, NB, drain, 0)
        pltpu.sync_copy(cntv, cnt_out.at[pl.ds(w * NB, NB)])

    return k(src, dst)


def _attn_layer_fn(relu):
    @functools.partial(
        pl.kernel,
        out_type=(
            jax.ShapeDtypeStruct((NB * BW, HC), jnp.float32),
            jax.ShapeDtypeStruct((ALEN,), jnp.float32),
        ),
        mesh=_MESH,
        compiler_params=_SC_PARAMS,
        scratch_types=[
            pltpu.VMEM((ECH,), jnp.int32),
            pltpu.VMEM((ECH,), jnp.int32),
            pltpu.VMEM((ECH,), jnp.int32),
            pltpu.VMEM((ECH,), jnp.int32),
            pltpu.VMEM((ECH, HC), jnp.float32),
            pltpu.VMEM((ECH, HC), jnp.float32),
            pltpu.VMEM((ECH, HC), jnp.float32),
            pltpu.VMEM((ECH * 4,), jnp.float32),
            pltpu.VMEM((ECH * 4,), jnp.float32),
            pltpu.VMEM((ECH, YW), jnp.float32),
            pltpu.VMEM((BW * 4 + 16,), jnp.float32),
            pltpu.VMEM((NT * NB,), jnp.int32),
            pltpu.VMEM_SHARED((NS * RSTR, YW), jnp.float32),
            pltpu.SemaphoreType.DMA,
            pltpu.SemaphoreType.DMA,
            pltpu.SemaphoreType.DMA,
        ],
    )
    def k(qn_h, kn_h, vn_h, skipn_h, etab_h, eid_h, src_h, dst_h, cnt_h,
          out_h, alpha_h,
          eidb, srcb, dstb, dstlb, qvb, kb, eb, alphab, exb, contrib,
          amax, cntsv, ysh, sem1, sem2, sem3):
        cidx = lax.axis_index("c")
        sidx = lax.axis_index("s")
        w = sidx * NC + cidx
        iota = lax.iota(jnp.int32, 16)
        m0 = iota == 0
        i4f = iota // 4
        i4m = iota - i4f * 4

        def _sget(ref, i):
            return plsc.load_gather(ref, [jnp.full((16,), i, jnp.int32)])[0]

        def _sputf(ref, i, v):
            plsc.store_scatter(ref, [jnp.full((16,), i, jnp.int32)],
                               jnp.full((16,), v), mask=m0)

        pltpu.sync_copy(cnt_h, cntsv)

        def _round(bkt):
            base_n = bkt * BW
            nloc = jnp.minimum(BW, 50000 - base_n)

            def gs_b(b, acc):
                def gs_t(t, a):
                    return a + ((_sget(cntsv, t * NB + b) + ECH - 1) // ECH) * ECH

                tot = lax.fori_loop(0, NT, gs_t, 0)
                return acc + jnp.where(b < bkt, tot, 0)

            gstart = lax.fori_loop(0, NB, gs_b, 0)

            def zc(r, _):
                for qq in range(YW // 16):
                    contrib[r, pl.ds(qq * 16, 16)] = jnp.zeros((16,), jnp.float32)
                return 0

            lax.fori_loop(0, ECH, zc, 0)

            def zy(cix, _):
                pltpu.sync_copy(
                    contrib,
                    ysh.at[pl.ds(
                        sidx * RSTR + pl.multiple_of(cix * ECH, ECH), ECH)])
                return 0

            lax.fori_loop(0, BW // ECH, zy, 0)

            def zy2(cix, _):
                pltpu.sync_copy(
                    contrib.at[pl.ds(0, 16)],
                    ysh.at[pl.ds(
                        sidx * RSTR
                        + pl.multiple_of((BW // ECH) * ECH + cix * 16, 16), 16)])
                return 0

            lax.fori_loop(0, (BW % ECH) // 16, zy2, 0)

            neg = jnp.full((16,), -3.0e38, jnp.float32)

            def za(g, _):
                amax[pl.ds(pl.multiple_of(g * 16, 16), 16)] = neg
                return 0

            lax.fori_loop(0, (BW * 4 + 16) // 16, za, 0)

            def p1_t(t, done):
                cnt = _sget(cntsv, t * NB + bkt)
                nch = (cnt + ECH - 1) // ECH

                def p1_c(ch, _):
                    off = pl.multiple_of(ch * ECH, ECH)
                    lv = jnp.minimum(ECH, cnt - off)
                    pltpu.sync_copy(eid_h.at[t, bkt, pl.ds(off, ECH)], eidb)
                    pltpu.sync_copy(src_h.at[t, bkt, pl.ds(off, ECH)], srcb)
                    pltpu.sync_copy(dst_h.at[t, bkt, pl.ds(off, ECH)], dstb)

                    def san(g, _):
                        ssl = pl.ds(pl.multiple_of(g * 16, 16), 16)
                        msk = (g * 16 + iota) < lv
                        eidb[ssl] = jnp.where(msk, eidb[ssl], 0)
                        srcb[ssl] = jnp.where(msk, srcb[ssl], 0)
                        dstb[ssl] = jnp.where(msk, dstb[ssl], base_n)
                        return 0

                    lax.fori_loop(0, ECH // 16, san, 0)

                    c1 = pltpu.async_copy(qn_h.at[dstb], qvb, sem1)
                    c2 = pltpu.async_copy(kn_h.at[srcb], kb, sem2)
                    c3 = pltpu.async_copy(etab_h.at[eidb], eb, sem3)
                    c1.wait()
                    c2.wait()
                    c3.wait()

                    m4 = iota < 4
                    i44 = iota & 3

                    def p1_e(j, _):
                        dl = _sget(dstb, j) - base_n
                        sv = jnp.zeros((16,), jnp.float32)
                        for h in range(HEADS):
                            sl = pl.ds(h * CH, CH)
                            s_h = jnp.sum(
                                qvb[j, sl] * (kb[j, sl] + eb[j, sl])) * 0.25
                            sv = jnp.where(iota == h, s_h, sv)
                        ai = dl * 4 + i44
                        mg = plsc.load_gather(amax, [ai])
                        plsc.store_scatter(amax, [ai], jnp.maximum(mg, sv),
                                           mask=m4)
                        plsc.store_scatter(alphab, [j * 4 + i44], sv, mask=m4)
                        return 0

                    lax.fori_loop(0, lv, p1_e, 0)
                    pltpu.sync_copy(
                        alphab,
                        alpha_h.at[pl.ds(
                            pl.multiple_of((gstart + done + off) * 4, ECH * 4),
                            ECH * 4)])
                    return 0

                lax.fori_loop(0, nch, p1_c, 0)
                return done + nch * ECH

            lax.fori_loop(0, NT, p1_t, 0)

            def p2_t(t, done):
                cnt = _sget(cntsv, t * NB + bkt)
                nch = (cnt + ECH - 1) // ECH

                def p2_c(ch, _):
                    off = pl.multiple_of(ch * ECH, ECH)
                    lv = jnp.minimum(ECH, cnt - off)
                    pltpu.sync_copy(eid_h.at[t, bkt, pl.ds(off, ECH)], eidb)
                    pltpu.sync_copy(src_h.at[t, bkt, pl.ds(off, ECH)], srcb)
                    pltpu.sync_copy(dst_h.at[t, bkt, pl.ds(off, ECH)], dstb)

                    def san(g, _):
                        ssl = pl.ds(pl.multiple_of(g * 16, 16), 16)
                        msk = (g * 16 + iota) < lv
                        eidb[ssl] = jnp.where(msk, eidb[ssl], 0)
                        srcb[ssl] = jnp.where(msk, srcb[ssl], 0)
                        dstb[ssl] = jnp.where(msk, dstb[ssl], base_n)
                        return 0

                    lax.fori_loop(0, ECH // 16, san, 0)

                    c1 = pltpu.async_copy(vn_h.at[srcb], qvb, sem1)
                    c2 = pltpu.async_copy(etab_h.at[eidb], eb, sem2)
                    pltpu.sync_copy(
                        alpha_h.at[pl.ds(
                            pl.multiple_of((gstart + done + off) * 4, ECH * 4),
                            ECH * 4)], alphab)

                    def mkdl(g, _):
                        sl = pl.ds(pl.multiple_of(g * 16, 16), 16)
                        lanes = g * 16 + iota
                        dv = dstb[sl] - base_n
                        dstlb[sl] = sidx * RSTR + jnp.where(
                            lanes < lv, dv, DUMP)
                        return 0

                    lax.fori_loop(0, ECH // 16, mkdl, 0)

                    def mkex(g, _):
                        ev = plsc.load_gather(dstlb, [g * 4 + i4f]) - sidx * RSTR
                        av = alphab[pl.ds(pl.multiple_of(g * 16, 16), 16)]
                        mg = plsc.load_gather(amax, [ev * 4 + i4m])
                        exv = jnp.where(ev < DUMP, jnp.exp(av - mg), 0.0)
                        exb[pl.ds(pl.multiple_of(g * 16, 16), 16)] = exv
                        plsc.store_scatter(contrib, [g * 4 + i4f, 64 + i4m], exv)
                        return 0

                    lax.fori_loop(0, ECH * 4 // 16, mkex, 0)
                    c1.wait()
                    c2.wait()

                    def p2_e(j, _):
                        exq = plsc.load_gather(exb, [j * 4 + (iota & 3)])
                        for h in range(HEADS):
                            sl = pl.ds(h * CH, CH)
                            contrib[j, sl] = (qvb[j, sl] + eb[j, sl]) * exq[h]
                        return 0

                    lax.fori_loop(0, lv, p2_e, 0)
                    pltpu.sync_copy(contrib, ysh.at[dstlb], add=True)
                    return 0

                lax.fori_loop(0, nch, p2_c, 0)
                return done + nch * ECH

            lax.fori_loop(0, NT, p2_t, 0)

            def p3_n(nn, _):
                denv = contrib[nn, pl.ds(64, 16)]
                rv = jnp.where(denv != 0.0, 1.0 / denv, 0.0)
                for h in range(HEADS):
                    sl = pl.ds(h * CH, CH)
                    ov = contrib[nn, sl] * rv[h] + kb[nn, sl]
                    if relu:
                        ov = jnp.maximum(ov, 0.0)
                    eb[nn, sl] = ov
                return 0

            nfull = nloc // ECH
            ntail = (nloc - nfull * ECH) // 16

            def p3_f(cix, _):
                off = pl.multiple_of(cix * ECH, ECH)
                pltpu.sync_copy(ysh.at[pl.ds(sidx * RSTR + off, ECH)], contrib)
                pltpu.sync_copy(skipn_h.at[pl.ds(base_n + off, ECH)], kb)
                lax.fori_loop(0, ECH, p3_n, 0)
                pltpu.sync_copy(eb, out_h.at[pl.ds(base_n + off, ECH)])
                return 0

            lax.fori_loop(0, nfull, p3_f, 0)

            def p3_t(cix, _):
                off = pl.multiple_of(nfull * ECH + cix * 16, 16)
                pltpu.sync_copy(ysh.at[pl.ds(sidx * RSTR + off, 16)],
                                contrib.at[pl.ds(0, 16)])
                pltpu.sync_copy(skipn_h.at[pl.ds(base_n + off, 16)],
                                kb.at[pl.ds(0, 16)])
                lax.fori_loop(0, 16, p3_n, 0)
                pltpu.sync_copy(eb.at[pl.ds(0, 16)],
                                out_h.at[pl.ds(base_n + off, 16)])
                return 0

            lax.fori_loop(0, ntail, p3_t, 0)

        for rnd in range(NRND):
            _round(w + NT * rnd)

    return k


_attn_relu = _attn_layer_fn(True)
_attn_plain = _attn_layer_fn(False)


def _mm_kernel(x_ref, w_ref, b_ref, o_ref):
    o_ref[...] = (
        jnp.dot(x_ref[...], w_ref[...], preferred_element_type=jnp.float32)
        + b_ref[...]
    )


def _project(x, W, b, blk):
    M, K = x.shape
    N = W.shape[1]
    assert M % blk == 0, (M, blk)
    return pl.pallas_call(
        _mm_kernel,
        grid=(M // blk,),
        in_specs=[
            pl.BlockSpec((blk, K), lambda i: (i, 0)),
            pl.BlockSpec((K, N), lambda i: (0, 0)),
            pl.BlockSpec((1, N), lambda i: (0, 0)),
        ],
        out_specs=pl.BlockSpec((blk, N), lambda i: (i, 0)),
        out_shape=jax.ShapeDtypeStruct((M, N), jnp.float32),
    )(x, W, b.reshape(1, N))


def _mm_split_kernel(nout, x_ref, w_ref, b_ref, *o_refs):
    acc = (
        jnp.dot(x_ref[...], w_ref[...], preferred_element_type=jnp.float32)
        + b_ref[...]
    )
    for i, o in enumerate(o_refs):
        o[...] = acc[:, i * HC:(i + 1) * HC]


def _project_split(x, Ws, bs, blk):
    M, K = x.shape
    nout = len(Ws)
    W = jnp.concatenate(Ws, axis=1)
    b = jnp.concatenate(bs).reshape(1, nout * HC)
    assert M % blk == 0, (M, blk)
    return pl.pallas_call(
        functools.partial(_mm_split_kernel, nout),
        grid=(M // blk,),
        in_specs=[
            pl.BlockSpec((blk, K), lambda i: (i, 0)),
            pl.BlockSpec((K, nout * HC), lambda i: (0, 0)),
            pl.BlockSpec((1, nout * HC), lambda i: (0, 0)),
        ],
        out_specs=[
            pl.BlockSpec((blk, HC), lambda i: (i, 0)) for _ in range(nout)
        ],
        out_shape=[
            jax.ShapeDtypeStruct((M, HC), jnp.float32) for _ in range(nout)
        ],
    )(x, W, b)


def _proj_nodes(x, p):
    return _project_split(
        x,
        [p["q"]["W"], p["k"]["W"], p["v"]["W"], p["skip"]["W"]],
        [p["q"]["b"], p["k"]["b"], p["v"]["b"], p["skip"]["b"]],
        blk=512,
    )


def kernel(memory, n_id, edge_index, edge_attr, params1, params2):
    N = n_id.shape[0]
    nid_pad = jnp.pad(n_id.astype(jnp.int32), (0, NT * 1664 - N))
    x = _gather_rows(memory, nid_pad)[: NB * BW]
    src = edge_index[0].astype(jnp.int32)
    dst = edge_index[1].astype(jnp.int32)
    eid_r, src_r, dst_r, counts = _bucket_edges(src, dst)
    e1, e2 = _project_split(
        edge_attr,
        [params1["e"]["W"], params2["e"]["W"]],
        [params1["e"]["b"], params2["e"]["b"]],
        blk=2000,
    )
    qn, kn, vn, sk = _proj_nodes(x, params1)
    x2, _ = _attn_relu(qn, kn, vn, sk, e1, eid_r, src_r, dst_r, counts)
    qn2, kn2, vn2, sk2 = _proj_nodes(x2, params2)
    x3, _ = _attn_plain(qn2, kn2, vn2, sk2, e2, eid_r, src_r, dst_r, counts)
    return x3[:N]


# speedup vs baseline: 13.6483x; 1.0031x over previous
import functools

import jax
import jax.numpy as jnp
import numpy as np
from jax import lax
from jax.experimental import pallas as pl
from jax.experimental.pallas import tpu as pltpu
from jax.experimental.pallas import tpu_sc as plsc

HEADS = 4
CH = 16
HC = HEADS * CH

NC, NS = 2, 16
NT = NC * NS

_MESH = plsc.VectorSubcoreMesh(
    core_axis_name="c", subcore_axis_name="s", num_cores=NC, num_subcores=NS
)
_SC_PARAMS = pltpu.CompilerParams(
    use_tc_tiling_on_sc=False, needs_layout_passes=False
)


def _wid():
    return lax.axis_index("s") * NC + lax.axis_index("c")


def _gather_rows(table, idx, rows_per_tile=1664, chunk=128):
    B = idx.shape[0]
    D = table.shape[1]
    assert B == NT * rows_per_tile and rows_per_tile % chunk == 0

    @functools.partial(
        pl.kernel,
        out_type=jax.ShapeDtypeStruct((B, D), jnp.float32),
        mesh=_MESH,
        compiler_params=_SC_PARAMS,
        scratch_types=[
            pltpu.VMEM((chunk,), jnp.int32),
            pltpu.VMEM((chunk, D), jnp.float32),
            pltpu.SemaphoreType.DMA,
        ],
    )
    def k(table_hbm, idx_hbm, out_hbm, idx_v, rows_v, sem):
        base = _wid() * rows_per_tile

        def body(c, _):
            off = base + c * chunk
            pltpu.sync_copy(idx_hbm.at[pl.ds(off, chunk)], idx_v)
            pltpu.async_copy(table_hbm.at[idx_v], rows_v, sem).wait()
            pltpu.sync_copy(rows_v, out_hbm.at[pl.ds(off, chunk)])
            return 0

        lax.fori_loop(0, rows_per_tile // chunk, body, 0)

    return k(table, idx)


E = 800000
NRND = 2
NB = NT * NRND
BW = 784
ES = E // NT
CAP = 25088
ECH = 128
YW = 80
DUMP = BW
RSTR = BW + 16
ALEN = (E + NT * NB * ECH) * 4


def _bucket_edges(src, dst):
    @functools.partial(
        pl.kernel,
        out_type=(
            jax.ShapeDtypeStruct((NT, NB, CAP), jnp.int32),
            jax.ShapeDtypeStruct((NT, NB, CAP), jnp.int32),
            jax.ShapeDtypeStruct((NT, NB, CAP), jnp.int32),
            jax.ShapeDtypeStruct((NT * NB,), jnp.int32),
        ),
        mesh=_MESH,
        compiler_params=_SC_PARAMS,
        scratch_types=[
            pltpu.VMEM((1000,), jnp.int32),
            pltpu.VMEM((1000,), jnp.int32),
            pltpu.VMEM((NB * 64,), jnp.int32),
            pltpu.VMEM((NB * 64,), jnp.int32),
            pltpu.VMEM((NB * 64,), jnp.int32),
            pltpu.VMEM((NB,), jnp.int32),
        ],
    )
    def k(src_hbm, dst_hbm, eid_out, src_out, dst_out, cnt_out,
          sbuf, dbuf, st_e, st_s, st_d, cntv):
        w = _wid()
        base_e = w * ES
        iota = lax.iota(jnp.int32, 16)
        m0 = iota == 0

        def _sget(ref, i):
            return plsc.load_gather(ref, [jnp.full((16,), i, jnp.int32)])[0]

        def _sput(ref, i, v):
            plsc.store_scatter(ref, [jnp.full((16,), i, jnp.int32)],
                               jnp.full((16,), v), mask=m0)

        def zb(g, _):
            cntv[pl.ds(g * 16, 16)] = jnp.zeros((16,), jnp.int32)
            return 0

        lax.fori_loop(0, NB // 16, zb, 0)

        def chunk(c, _):
            off = c * 1000
            pltpu.sync_copy(src_hbm.at[pl.ds(base_e + off, 1000)], sbuf)
            pltpu.sync_copy(dst_hbm.at[pl.ds(base_e + off, 1000)], dbuf)

            def edge(j, _):
                d = _sget(dbuf, j)
                b = d // BW
                cb = _sget(cntv, b)
                slot = cb & 63
                _sput(st_e, b * 64 + slot, base_e + off + j)
                _sput(st_s, b * 64 + slot, _sget(sbuf, j))
                _sput(st_d, b * 64 + slot, d)

                @pl.when(slot == 63)
                def _flush():
                    fo = pl.multiple_of(cb - 63, 64)
                    pltpu.sync_copy(st_e.at[pl.ds(b * 64, 64)],
                                    eid_out.at[w, b, pl.ds(fo, 64)])
                    pltpu.sync_copy(st_s.at[pl.ds(b * 64, 64)],
                                    src_out.at[w, b, pl.ds(fo, 64)])
                    pltpu.sync_copy(st_d.at[pl.ds(b * 64, 64)],
                                    dst_out.at[w, b, pl.ds(fo, 64)])

                _sput(cntv, b, cb + 1)
                return 0

            lax.fori_loop(0, 1000, edge, 0)
            return 0

        lax.fori_loop(0, ES // 1000, chunk, 0)

        def drain(b, _):
            cb = _sget(cntv, b)

            @pl.when((cb & 63) != 0)
            def _fl():
                fo = pl.multiple_of(cb - (cb & 63), 64)
                pltpu.sync_copy(st_e.at[pl.ds(b * 64, 64)],
                                eid_out.at[w, b, pl.ds(fo, 64)])
                pltpu.sync_copy(st_s.at[pl.ds(b * 64, 64)],
                                src_out.at[w, b, pl.ds(fo, 64)])
                pltpu.sync_copy(st_d.at[pl.ds(b * 64, 64)],
                                dst_out.at[w, b, pl.ds(fo, 64)])

            return 0

        lax.fori_loop(0, NB, drain, 0)
        pltpu.sync_copy(cntv, cnt_out.at[pl.ds(w * NB, NB)])

    return k(src, dst)


def _attn_layer_fn(relu):
    @functools.partial(
        pl.kernel,
        out_type=(
            jax.ShapeDtypeStruct((NB * BW, HC), jnp.float32),
            jax.ShapeDtypeStruct((ALEN,), jnp.float32),
        ),
        mesh=_MESH,
        compiler_params=_SC_PARAMS,
        scratch_types=[
            pltpu.VMEM((ECH,), jnp.int32),
            pltpu.VMEM((ECH,), jnp.int32),
            pltpu.VMEM((ECH,), jnp.int32),
            pltpu.VMEM((ECH,), jnp.int32),
            pltpu.VMEM((ECH, HC), jnp.float32),
            pltpu.VMEM((ECH, HC), jnp.float32),
            pltpu.VMEM((ECH, HC), jnp.float32),
            pltpu.VMEM((ECH * 4,), jnp.float32),
            pltpu.VMEM((ECH * 4,), jnp.float32),
            pltpu.VMEM((ECH, YW), jnp.float32),
            pltpu.VMEM((BW * 4 + 16,), jnp.float32),
            pltpu.VMEM((NT * NB,), jnp.int32),
            pltpu.VMEM_SHARED((NS * RSTR, YW), jnp.float32),
            pltpu.SemaphoreType.DMA,
            pltpu.SemaphoreType.DMA,
            pltpu.SemaphoreType.DMA,
            pltpu.SemaphoreType.DMA,
        ],
    )
    def k(qn_h, kn_h, vn_h, skipn_h, etab_h, eid_h, src_h, dst_h, cnt_h,
          out_h, alpha_h,
          eidb, srcb, dstb, dstlb, qvb, kb, eb, alphab, exb, contrib,
          amax, cntsv, ysh, sem1, sem2, sem3, sem4):
        cidx = lax.axis_index("c")
        sidx = lax.axis_index("s")
        w = sidx * NC + cidx
        iota = lax.iota(jnp.int32, 16)
        m0 = iota == 0
        i4f = iota // 4
        i4m = iota - i4f * 4

        def _sget(ref, i):
            return plsc.load_gather(ref, [jnp.full((16,), i, jnp.int32)])[0]

        def _sputf(ref, i, v):
            plsc.store_scatter(ref, [jnp.full((16,), i, jnp.int32)],
                               jnp.full((16,), v), mask=m0)

        pltpu.sync_copy(cnt_h, cntsv)

        def _round(bkt):
            base_n = bkt * BW
            nloc = jnp.minimum(BW, 50000 - base_n)

            def gs_b(b, acc):
                def gs_t(t, a):
                    return a + ((_sget(cntsv, t * NB + b) + ECH - 1) // ECH) * ECH

                tot = lax.fori_loop(0, NT, gs_t, 0)
                return acc + jnp.where(b < bkt, tot, 0)

            gstart = lax.fori_loop(0, NB, gs_b, 0)

            def zc(r, _):
                for qq in range(YW // 16):
                    contrib[r, pl.ds(qq * 16, 16)] = jnp.zeros((16,), jnp.float32)
                return 0

            lax.fori_loop(0, ECH, zc, 0)

            def zy(cix, _):
                pltpu.sync_copy(
                    contrib,
                    ysh.at[pl.ds(
                        sidx * RSTR + pl.multiple_of(cix * ECH, ECH), ECH)])
                return 0

            lax.fori_loop(0, BW // ECH, zy, 0)

            def zy2(cix, _):
                pltpu.sync_copy(
                    contrib.at[pl.ds(0, 16)],
                    ysh.at[pl.ds(
                        sidx * RSTR
                        + pl.multiple_of((BW // ECH) * ECH + cix * 16, 16), 16)])
                return 0

            lax.fori_loop(0, (BW % ECH) // 16, zy2, 0)

            neg = jnp.full((16,), -3.0e38, jnp.float32)

            def za(g, _):
                amax[pl.ds(pl.multiple_of(g * 16, 16), 16)] = neg
                return 0

            lax.fori_loop(0, (BW * 4 + 16) // 16, za, 0)

            def p1_t(t, done):
                cnt = _sget(cntsv, t * NB + bkt)
                nch = (cnt + ECH - 1) // ECH

                def p1_c(ch, _):
                    off = pl.multiple_of(ch * ECH, ECH)
                    lv = jnp.minimum(ECH, cnt - off)
                    l1 = pltpu.async_copy(
                        eid_h.at[t, bkt, pl.ds(off, ECH)], eidb, sem1)
                    l2 = pltpu.async_copy(
                        src_h.at[t, bkt, pl.ds(off, ECH)], srcb, sem2)
                    l3 = pltpu.async_copy(
                        dst_h.at[t, bkt, pl.ds(off, ECH)], dstb, sem3)
                    l1.wait()
                    l2.wait()
                    l3.wait()

                    def san(g, _):
                        ssl = pl.ds(pl.multiple_of(g * 16, 16), 16)
                        msk = (g * 16 + iota) < lv
                        eidb[ssl] = jnp.where(msk, eidb[ssl], 0)
                        srcb[ssl] = jnp.where(msk, srcb[ssl], 0)
                        dstb[ssl] = jnp.where(msk, dstb[ssl], base_n)
                        return 0

                    lax.fori_loop(0, ECH // 16, san, 0)

                    c1 = pltpu.async_copy(qn_h.at[dstb], qvb, sem1)
                    c2 = pltpu.async_copy(kn_h.at[srcb], kb, sem2)
                    c3 = pltpu.async_copy(etab_h.at[eidb], eb, sem3)
                    c1.wait()
                    c2.wait()
                    c3.wait()

                    m4 = iota < 4
                    i44 = iota & 3

                    def p1_e(j, _):
                        dl = _sget(dstb, j) - base_n
                        sv = jnp.zeros((16,), jnp.float32)
                        for h in range(HEADS):
                            sl = pl.ds(h * CH, CH)
                            s_h = jnp.sum(
                                qvb[j, sl] * (kb[j, sl] + eb[j, sl])) * 0.25
                            sv = jnp.where(iota == h, s_h, sv)
                        ai = dl * 4 + i44
                        mg = plsc.load_gather(amax, [ai])
                        plsc.store_scatter(amax, [ai], jnp.maximum(mg, sv),
                                           mask=m4)
                        plsc.store_scatter(alphab, [j * 4 + i44], sv, mask=m4)
                        return 0

                    lax.fori_loop(0, lv, p1_e, 0)
                    pltpu.sync_copy(
                        alphab,
                        alpha_h.at[pl.ds(
                            pl.multiple_of((gstart + done + off) * 4, ECH * 4),
                            ECH * 4)])
                    return 0

                lax.fori_loop(0, nch, p1_c, 0)
                return done + nch * ECH

            lax.fori_loop(0, NT, p1_t, 0)

            def p2_t(t, done):
                cnt = _sget(cntsv, t * NB + bkt)
                nch = (cnt + ECH - 1) // ECH

                def p2_c(ch, _):
                    off = pl.multiple_of(ch * ECH, ECH)
                    lv = jnp.minimum(ECH, cnt - off)
                    l1 = pltpu.async_copy(
                        eid_h.at[t, bkt, pl.ds(off, ECH)], eidb, sem1)
                    l2 = pltpu.async_copy(
                        src_h.at[t, bkt, pl.ds(off, ECH)], srcb, sem2)
                    l3 = pltpu.async_copy(
                        dst_h.at[t, bkt, pl.ds(off, ECH)], dstb, sem3)
                    l1.wait()
                    l2.wait()
                    l3.wait()

                    def san(g, _):
                        ssl = pl.ds(pl.multiple_of(g * 16, 16), 16)
                        msk = (g * 16 + iota) < lv
                        eidb[ssl] = jnp.where(msk, eidb[ssl], 0)
                        srcb[ssl] = jnp.where(msk, srcb[ssl], 0)
                        dstb[ssl] = jnp.where(msk, dstb[ssl], base_n)
                        return 0

                    lax.fori_loop(0, ECH // 16, san, 0)

                    c1 = pltpu.async_copy(vn_h.at[srcb], qvb, sem1)
                    c2 = pltpu.async_copy(etab_h.at[eidb], eb, sem2)
                    c4 = pltpu.async_copy(
                        alpha_h.at[pl.ds(
                            pl.multiple_of((gstart + done + off) * 4, ECH * 4),
                            ECH * 4)], alphab, sem4)
                    c4.wait()

                    def mkdl(g, _):
                        sl = pl.ds(pl.multiple_of(g * 16, 16), 16)
                        lanes = g * 16 + iota
                        dv = dstb[sl] - base_n
                        dstlb[sl] = sidx * RSTR + jnp.where(
                            lanes < lv, dv, DUMP)
                        return 0

                    lax.fori_loop(0, ECH // 16, mkdl, 0)

                    def mkex(g, _):
                        ev = plsc.load_gather(dstlb, [g * 4 + i4f]) - sidx * RSTR
                        av = alphab[pl.ds(pl.multiple_of(g * 16, 16), 16)]
                        mg = plsc.load_gather(amax, [ev * 4 + i4m])
                        exv = jnp.where(ev < DUMP, jnp.exp(av - mg), 0.0)
                        exb[pl.ds(pl.multiple_of(g * 16, 16), 16)] = exv
                        plsc.store_scatter(contrib, [g * 4 + i4f, 64 + i4m], exv)
                        return 0

                    lax.fori_loop(0, ECH * 4 // 16, mkex, 0)
                    c1.wait()
                    c2.wait()

                    def p2_e(j, _):
                        exq = plsc.load_gather(exb, [j * 4 + (iota & 3)])
                        for h in range(HEADS):
                            sl = pl.ds(h * CH, CH)
                            contrib[j, sl] = (qvb[j, sl] + eb[j, sl]) * exq[h]
                        return 0

                    lax.fori_loop(0, lv, p2_e, 0)
                    pltpu.sync_copy(contrib, ysh.at[dstlb], add=True)
                    return 0

                lax.fori_loop(0, nch, p2_c, 0)
                return done + nch * ECH

            lax.fori_loop(0, NT, p2_t, 0)

            def p3_n(nn, _):
                denv = contrib[nn, pl.ds(64, 16)]
                rv = jnp.where(denv != 0.0, 1.0 / denv, 0.0)
                for h in range(HEADS):
                    sl = pl.ds(h * CH, CH)
                    ov = contrib[nn, sl] * rv[h] + kb[nn, sl]
                    if relu:
                        ov = jnp.maximum(ov, 0.0)
                    eb[nn, sl] = ov
                return 0

            nfull = nloc // ECH
            ntail = (nloc - nfull * ECH) // 16

            def p3_f(cix, _):
                off = pl.multiple_of(cix * ECH, ECH)
                pltpu.sync_copy(ysh.at[pl.ds(sidx * RSTR + off, ECH)], contrib)
                pltpu.sync_copy(skipn_h.at[pl.ds(base_n + off, ECH)], kb)
                lax.fori_loop(0, ECH, p3_n, 0)
                pltpu.sync_copy(eb, out_h.at[pl.ds(base_n + off, ECH)])
                return 0

            lax.fori_loop(0, nfull, p3_f, 0)

            def p3_t(cix, _):
                off = pl.multiple_of(nfull * ECH + cix * 16, 16)
                pltpu.sync_copy(ysh.at[pl.ds(sidx * RSTR + off, 16)],
                                contrib.at[pl.ds(0, 16)])
                pltpu.sync_copy(skipn_h.at[pl.ds(base_n + off, 16)],
                                kb.at[pl.ds(0, 16)])
                lax.fori_loop(0, 16, p3_n, 0)
                pltpu.sync_copy(eb.at[pl.ds(0, 16)],
                                out_h.at[pl.ds(base_n + off, 16)])
                return 0

            lax.fori_loop(0, ntail, p3_t, 0)

        for rnd in range(NRND):
            _round(w + NT * rnd)

    return k


_attn_relu = _attn_layer_fn(True)
_attn_plain = _attn_layer_fn(False)


def _mm_kernel(x_ref, w_ref, b_ref, o_ref):
    o_ref[...] = (
        jnp.dot(x_ref[...], w_ref[...], preferred_element_type=jnp.float32)
        + b_ref[...]
    )


def _project(x, W, b, blk):
    M, K = x.shape
    N = W.shape[1]
    assert M % blk == 0, (M, blk)
    return pl.pallas_call(
        _mm_kernel,
        grid=(M // blk,),
        in_specs=[
            pl.BlockSpec((blk, K), lambda i: (i, 0)),
            pl.BlockSpec((K, N), lambda i: (0, 0)),
            pl.BlockSpec((1, N), lambda i: (0, 0)),
        ],
        out_specs=pl.BlockSpec((blk, N), lambda i: (i, 0)),
        out_shape=jax.ShapeDtypeStruct((M, N), jnp.float32),
    )(x, W, b.reshape(1, N))


def _mm_split_kernel(nout, x_ref, w_ref, b_ref, *o_refs):
    acc = (
        jnp.dot(x_ref[...], w_ref[...], preferred_element_type=jnp.float32)
        + b_ref[...]
    )
    for i, o in enumerate(o_refs):
        o[...] = acc[:, i * HC:(i + 1) * HC]


def _project_split(x, Ws, bs, blk):
    M, K = x.shape
    nout = len(Ws)
    W = jnp.concatenate(Ws, axis=1)
    b = jnp.concatenate(bs).reshape(1, nout * HC)
    assert M % blk == 0, (M, blk)
    return pl.pallas_call(
        functools.partial(_mm_split_kernel, nout),
        grid=(M // blk,),
        in_specs=[
            pl.BlockSpec((blk, K), lambda i: (i, 0)),
            pl.BlockSpec((K, nout * HC), lambda i: (0, 0)),
            pl.BlockSpec((1, nout * HC), lambda i: (0, 0)),
        ],
        out_specs=[
            pl.BlockSpec((blk, HC), lambda i: (i, 0)) for _ in range(nout)
        ],
        out_shape=[
            jax.ShapeDtypeStruct((M, HC), jnp.float32) for _ in range(nout)
        ],
    )(x, W, b)


def _proj_nodes(x, p):
    return _project_split(
        x,
        [p["q"]["W"], p["k"]["W"], p["v"]["W"], p["skip"]["W"]],
        [p["q"]["b"], p["k"]["b"], p["v"]["b"], p["skip"]["b"]],
        blk=512,
    )


def kernel(memory, n_id, edge_index, edge_attr, params1, params2):
    N = n_id.shape[0]
    nid_pad = jnp.pad(n_id.astype(jnp.int32), (0, NT * 1664 - N))
    x = _gather_rows(memory, nid_pad)[: NB * BW]
    src = edge_index[0].astype(jnp.int32)
    dst = edge_index[1].astype(jnp.int32)
    eid_r, src_r, dst_r, counts = _bucket_edges(src, dst)
    e1, e2 = _project_split(
        edge_attr,
        [params1["e"]["W"], params2["e"]["W"]],
        [params1["e"]["b"], params2["e"]["b"]],
        blk=2000,
    )
    qn, kn, vn, sk = _proj_nodes(x, params1)
    x2, _ = _attn_relu(qn, kn, vn, sk, e1, eid_r, src_r, dst_r, counts)
    qn2, kn2, vn2, sk2 = _proj_nodes(x2, params2)
    x3, _ = _attn_plain(qn2, kn2, vn2, sk2, e2, eid_r, src_r, dst_r, counts)
    return x3[:N]
